# Initial kernel scaffold; baseline (speedup 1.0000x reference)
import jax
import jax.numpy as jnp
from jax.experimental import pallas as pl


def kernel(inputs, old_activation, input_fields, sub_rows, sub_cols, sub_vals, sup_rows, sup_cols, sup_vals):
    raise NotImplementedError("write your pallas kernel here")


# speedup vs baseline: 2.4952x; 2.4952x over previous
import dataclasses
import functools

import jax
import jax.numpy as jnp
from jax import lax
from jax.experimental import pallas as pl
from jax.experimental.pallas import tpu as pltpu
from jax.experimental.pallas import tpu_sc as plsc

NUM_DATA = 10000
INPUT_DIM = 128
B = 5000

NW = 32
ROWS_PER_W = 320
OUT_PAD = NW * ROWS_PER_W
K = 128
DIFF_CHUNK = 160
DIFF_HALF = 80
B_PAD = NW * DIFF_CHUNK

_MESH = plsc.VectorSubcoreMesh(core_axis_name="c", subcore_axis_name="s")

_CP = pltpu.CompilerParams()
if "needs_layout_passes" in pltpu.CompilerParams.__dataclass_fields__:
    _CP = dataclasses.replace(_CP, needs_layout_passes=False)


def _iota16():
    return lax.iota(jnp.int32, 16)


def _lane_i32(vec, lane_mask):
    return jnp.sum(jnp.where(lane_mask, vec, 0))


def _lane_f32(vec, lane_mask):
    return jnp.sum(jnp.where(lane_mask, vec, jnp.float32(0)))


def _diff_body(old_hbm, inputs_hbm, if_hbm, diff_hbm, idx_v, g_v, in_v, sem):
    c = lax.axis_index("c")
    s = lax.axis_index("s")
    wid = c * 16 + s
    base = wid * DIFF_CHUNK
    for h in range(DIFF_CHUNK // DIFF_HALF):
        b2 = base + h * DIFF_HALF
        pltpu.sync_copy(if_hbm.at[pl.ds(b2, DIFF_HALF)], idx_v)
        pltpu.async_copy(old_hbm.at[idx_v], g_v, sem).wait()
        pltpu.sync_copy(inputs_hbm.at[pl.ds(b2, DIFF_HALF)], in_v)

        @pl.loop(0, DIFF_HALF)
        def _(r):
            for cc in range(INPUT_DIM // 16):
                sl = pl.ds(cc * 16, 16)
                g_v[r, sl] = in_v[r, sl] - g_v[r, sl]

        pltpu.sync_copy(g_v, diff_hbm.at[pl.ds(b2, DIFF_HALF)])


@jax.jit
def _diff_kernel(old_act, inputs_p, if_p):
    return pl.kernel(
        _diff_body,
        out_type=jax.ShapeDtypeStruct((B_PAD, INPUT_DIM), jnp.float32),
        mesh=_MESH,
        compiler_params=_CP,
        scratch_types=[
            pltpu.VMEM((DIFF_HALF,), jnp.int32),
            pltpu.VMEM((DIFF_HALF, INPUT_DIM), jnp.float32),
            pltpu.VMEM((DIFF_HALF, INPUT_DIM), jnp.float32),
            pltpu.SemaphoreType.DMA,
        ],
    )(old_act, inputs_p, if_p)


def _spmm_body(diff_hbm, old_hbm,
               subr_hbm, subc_hbm, subv_hbm,
               supr_hbm, supc_hbm, supv_hbm,
               bounds_hbm, out_hbm,
               bounds_v, rows_v, cols_v, vals_v, g_v, acc_v, sem):
    c = lax.axis_index("c")
    s = lax.axis_index("s")
    wid = c * 16 + s
    row0 = wid * ROWS_PER_W
    iota = _iota16()
    lane_s = iota == s

    zero16 = jnp.zeros((16,), jnp.float32)

    @pl.loop(0, ROWS_PER_W)
    def _(r):
        for cc in range(INPUT_DIM // 16):
            acc_v[r, pl.ds(cc * 16, 16)] = zero16

    pltpu.sync_copy(bounds_hbm, bounds_v)

    def do_coo(coo, rows_hbm, cols_hbm, vals_hbm, x_hbm):
        vec_lo = bounds_v[coo * 4 + c, :]
        vec_hi = bounds_v[coo * 4 + 2 + c, :]
        e_lo = _lane_i32(vec_lo, lane_s)
        e_hi = _lane_i32(vec_hi, lane_s)
        e0 = pl.multiple_of((e_lo // 8) * 8, 8)
        nb = (e_hi - e0 + K - 1) // K

        def batch(b, carry):
            base = pl.multiple_of(e0 + b * K, 8)
            pltpu.sync_copy(rows_hbm.at[pl.ds(base, K)], rows_v)
            pltpu.sync_copy(cols_hbm.at[pl.ds(base, K)], cols_v)
            pltpu.sync_copy(vals_hbm.at[pl.ds(base, K)], vals_v)
            pltpu.async_copy(x_hbm.at[cols_v], g_v, sem).wait()

            @pl.loop(0, K, step=16)
            def _(k0):
                vrows = rows_v[pl.ds(k0, 16)]
                vvals = vals_v[pl.ds(k0, 16)]
                ge = base + k0 + iota
                valid = (ge >= e_lo) & (ge < e_hi)
                veff = jnp.where(valid, vvals, jnp.float32(0))
                lrow = jnp.clip(vrows - row0, 0, ROWS_PER_W - 1)
                for l in range(16):
                    lm = iota == l
                    val_l = _lane_f32(veff, lm)
                    row_l = _lane_i32(lrow, lm)
                    for cc in range(INPUT_DIM // 16):
                        sl = pl.ds(cc * 16, 16)
                        g = g_v[k0 + l, sl]
                        plsc.addupdate(acc_v.at[row_l, sl], val_l * g)

            return carry

        lax.fori_loop(0, nb, batch, 0)

    do_coo(0, subr_hbm, subc_hbm, subv_hbm, diff_hbm)
    do_coo(1, supr_hbm, supc_hbm, supv_hbm, old_hbm)

    pltpu.sync_copy(acc_v, out_hbm.at[pl.ds(row0, ROWS_PER_W)])


@jax.jit
def _spmm_kernel(diff, old_act, subr, subc, subv, supr, supc, supv, bounds):
    return pl.kernel(
        _spmm_body,
        out_type=jax.ShapeDtypeStruct((OUT_PAD, INPUT_DIM), jnp.float32),
        mesh=_MESH,
        compiler_params=_CP,
        scratch_types=[
            pltpu.VMEM((8, 16), jnp.int32),
            pltpu.VMEM((K,), jnp.int32),
            pltpu.VMEM((K,), jnp.int32),
            pltpu.VMEM((K,), jnp.float32),
            pltpu.VMEM((K, INPUT_DIM), jnp.float32),
            pltpu.VMEM((ROWS_PER_W, INPUT_DIM), jnp.float32),
            pltpu.SemaphoreType.DMA,
        ],
    )(diff, old_act, subr, subc, subv, supr, supc, supv, bounds)


def kernel(inputs, old_activation, input_fields, sub_rows, sub_cols, sub_vals,
           sup_rows, sup_cols, sup_vals):
    i32 = jnp.int32
    input_fields = input_fields.astype(i32)
    sub_rows = sub_rows.astype(i32)
    sub_cols = sub_cols.astype(i32)
    sup_rows = sup_rows.astype(i32)
    sup_cols = sup_cols.astype(i32)

    inputs_p = jnp.pad(inputs, ((0, B_PAD - B), (0, 0)))
    if_p = jnp.pad(input_fields, (0, B_PAD - B))
    subr_p = jnp.pad(sub_rows, (0, K))
    subc_p = jnp.pad(sub_cols, (0, K))
    subv_p = jnp.pad(sub_vals, (0, K))
    supr_p = jnp.pad(sup_rows, (0, K))
    supc_p = jnp.pad(sup_cols, (0, K))
    supv_p = jnp.pad(sup_vals, (0, K))

    row_starts = jnp.arange(NW + 1, dtype=i32) * ROWS_PER_W
    bs = jnp.searchsorted(sub_rows, row_starts).astype(i32)
    bp = jnp.searchsorted(sup_rows, row_starts).astype(i32)
    bounds = jnp.concatenate(
        [bs[:NW].reshape(2, 16), bs[1:NW + 1].reshape(2, 16),
         bp[:NW].reshape(2, 16), bp[1:NW + 1].reshape(2, 16)], axis=0)

    diff = _diff_kernel(old_activation, inputs_p, if_p)
    out_p = _spmm_kernel(diff, old_activation, subr_p, subc_p, subv_p,
                         supr_p, supc_p, supv_p, bounds)
    return out_p[:NUM_DATA]


# speedup vs baseline: 9.3344x; 3.7410x over previous
import dataclasses

import jax
import jax.numpy as jnp
from jax import lax
from jax.experimental import pallas as pl
from jax.experimental.pallas import tpu as pltpu
from jax.experimental.pallas import tpu_sc as plsc

NUM_DATA = 10000
INPUT_DIM = 128
B = 5000
NC = INPUT_DIM // 16

NW = 32
NS = 16
ROWS_PER_W = 320
OUT_PAD = NW * ROWS_PER_W
OLD_PAD = 10112
K = 128
DIFF_PER_S = 320
DIFF_WIN = 80
B_PAD = NS * DIFF_PER_S
E_SUB = 160000
E_SUP = 320000
EDGE_PAD = 5 * K

_MESH = plsc.VectorSubcoreMesh(core_axis_name="c", subcore_axis_name="s")

_CP = pltpu.CompilerParams()
if "needs_layout_passes" in pltpu.CompilerParams.__dataclass_fields__:
    _CP = dataclasses.replace(_CP, needs_layout_passes=False)


def _splat(vec, lane):
    idx = jnp.full((16, 1), lane, jnp.int32)
    dn = lax.GatherDimensionNumbers(
        offset_dims=(), collapsed_slice_dims=(0,), start_index_map=(0,))
    return lax.gather(vec, idx, dn, (1,),
                      mode=lax.GatherScatterMode.PROMISE_IN_BOUNDS)


def _lane_i32(vec, lane_mask):
    return jnp.sum(jnp.where(lane_mask, vec, 0))


def _lane_f32(vec, lane_mask):
    return jnp.sum(jnp.where(lane_mask, vec, jnp.float32(0)))


def _body(old_hbm, inputs_hbm, if_hbm,
          subr_hbm, subc_hbm, subv_hbm,
          supr_hbm, supc_hbm, supv_hbm,
          bounds_hbm, out_hbm, diff_hbm,
          bounds_v, idx_v, gin_v, din_v,
          rows0_v, cols0_v, vals0_v, rows1_v, cols1_v, vals1_v,
          g0_v, g1_v, acc_v,
          sem_i0, sem_i1, sem_g0, sem_g1, sem_s):
    c = lax.axis_index("c")
    s = lax.axis_index("s")
    wid = c * NS + s
    row0 = wid * ROWS_PER_W
    iota = lax.iota(jnp.int32, 16)
    lane_s = iota == s

    for h in range(DIFF_PER_S // DIFF_WIN):
        base = s * DIFF_PER_S + h * DIFF_WIN
        pltpu.sync_copy(if_hbm.at[pl.ds(base, DIFF_WIN)], idx_v)
        pltpu.async_copy(old_hbm.at[idx_v], gin_v, sem_g0).wait()
        pltpu.sync_copy(inputs_hbm.at[pl.ds(base, DIFF_WIN)], din_v)

        @pl.loop(0, DIFF_WIN)
        def _(r):
            for cc in range(NC):
                sl = pl.ds(cc * 16, 16)
                gin_v[r, sl] = din_v[r, sl] - gin_v[r, sl]

        pltpu.async_copy(gin_v, diff_hbm.at[pl.ds(c * B_PAD + base, DIFF_WIN)],
                         sem_s).wait()

    plsc.subcore_barrier()

    zero16 = jnp.zeros((16,), jnp.float32)

    @pl.loop(0, ROWS_PER_W)
    def _(r):
        for cc in range(NC):
            acc_v[r, pl.ds(cc * 16, 16)] = zero16

    pltpu.sync_copy(bounds_hbm, bounds_v)

    def do_coo(coo, rows_hbm, cols_hbm, vals_hbm, x_sh, e_max, col_off):
        vec_lo = bounds_v[coo * 4 + c, :]
        vec_hi = bounds_v[coo * 4 + 2 + c, :]
        e_lo = _lane_i32(vec_lo, lane_s)
        e_hi = _lane_i32(vec_hi, lane_s)
        e0 = pl.multiple_of((e_lo // 8) * 8, 8)
        nb = (e_hi - e0 + K - 1) // K
        nb2 = ((nb + 1) // 2) * 2

        bufs = ((rows0_v, cols0_v, vals0_v, g0_v, sem_i0, sem_g0),
                (rows1_v, cols1_v, vals1_v, g1_v, sem_i1, sem_g1))

        def base_of(b):
            return pl.multiple_of(jnp.minimum(e0 + b * K, e_max), 8)

        def issue_cols(b, buf):
            _, cols_v, _, _, sem_i, _ = buf
            pltpu.async_copy(cols_hbm.at[pl.ds(base_of(b), K)], cols_v, sem_i)

        def issue_rv(b, buf):
            rows_v, _, vals_v, _, _, sem_g = buf
            base = base_of(b)
            pltpu.async_copy(rows_hbm.at[pl.ds(base, K)],
                             rows_v.at[pl.ds(0, K)], sem_g)
            pltpu.async_copy(vals_hbm.at[pl.ds(base, K)],
                             vals_v.at[pl.ds(0, K)], sem_g)

        def wait_cols_issue_gather(buf):
            rows_v, cols_v, vals_v, g_v, sem_i, sem_g = buf
            pltpu.make_async_copy(cols_hbm.at[pl.ds(0, K)], cols_v, sem_i).wait()
            if col_off is not None:
                for cc in range(K // 16):
                    sl = pl.ds(cc * 16, 16)
                    cols_v[sl] = cols_v[sl] + col_off
            pltpu.async_copy(x_sh.at[cols_v], g_v, sem_i)

        def wait_gather(buf):
            rows_v, cols_v, vals_v, g_v, sem_i, sem_g = buf
            pltpu.make_async_copy(x_sh.at[cols_v], g_v, sem_i).wait()

        def wait_rv(buf):
            rows_v, cols_v, vals_v, g_v, sem_i, sem_g = buf
            pltpu.make_async_copy(rows_hbm.at[pl.ds(0, K)],
                                  rows_v.at[pl.ds(0, K)], sem_g).wait()
            pltpu.make_async_copy(vals_hbm.at[pl.ds(0, K)],
                                  vals_v.at[pl.ds(0, K)], sem_g).wait()

        def compute(b, buf):
            rows_v, cols_v, vals_v, g_v, _, sem_g = buf
            base = base_of(b)

            @plsc.parallel_loop(0, K, 8)
            def _(k0):
                vv = vals_v[pl.ds(k0, 16)]
                vr = rows_v[pl.ds(k0, 16)]
                ge = base + k0 + iota
                valid = (ge >= e_lo) & (ge < e_hi)
                veff = jnp.where(valid, vv, jnp.float32(0))
                lrow = jnp.clip(vr - row0, 0, ROWS_PER_W - 1)
                for l in range(8):
                    val_spl = _splat(veff, l)
                    row_spl = _splat(lrow, l)
                    gs = [g_v[k0 + l, pl.ds(cc * 16, 16)] for cc in range(NC)]
                    for cc in range(NC):
                        plsc.addupdate_scatter(
                            acc_v, [row_spl, iota + (cc * 16)],
                            val_spl * gs[cc])

        issue_cols(0, bufs[0])
        issue_rv(0, bufs[0])
        wait_cols_issue_gather(bufs[0])
        issue_cols(1, bufs[1])
        issue_rv(1, bufs[1])

        def pair(p, carry):
            b0 = p * 2
            for bi in range(2):
                b = b0 + bi
                cur, nxt = bufs[bi], bufs[1 - bi]
                wait_cols_issue_gather(nxt)
                wait_gather(cur)
                issue_cols(b + 2, cur)
                wait_rv(cur)
                compute(b, cur)
                issue_rv(b + 2, cur)
            return carry

        lax.fori_loop(0, nb2 // 2, pair, 0)
        wait_gather(bufs[0])
        pltpu.make_async_copy(cols_hbm.at[pl.ds(0, K)], cols1_v, sem_i1).wait()
        wait_rv(bufs[0])
        wait_rv(bufs[1])

    do_coo(1, supr_hbm, supc_hbm, supv_hbm, old_hbm,
           jnp.int32(E_SUP + EDGE_PAD - K), None)
    do_coo(0, subr_hbm, subc_hbm, subv_hbm, diff_hbm,
           jnp.int32(E_SUB + EDGE_PAD - K), c * B_PAD)

    pltpu.sync_copy(acc_v, out_hbm.at[pl.ds(row0, ROWS_PER_W)])


@jax.jit
def _fused_kernel(old_p, inputs_p, if_p, subr, subc, subv, supr, supc, supv,
                  bounds):
    return pl.kernel(
        _body,
        out_type=(jax.ShapeDtypeStruct((OUT_PAD, INPUT_DIM), jnp.float32),
                  jax.ShapeDtypeStruct((2 * B_PAD, INPUT_DIM), jnp.float32)),
        mesh=_MESH,
        compiler_params=_CP,
        scratch_types=[
            pltpu.VMEM((8, 16), jnp.int32),
            pltpu.VMEM((DIFF_WIN,), jnp.int32),
            pltpu.VMEM((DIFF_WIN, INPUT_DIM), jnp.float32),
            pltpu.VMEM((DIFF_WIN, INPUT_DIM), jnp.float32),
            pltpu.VMEM((K + 16,), jnp.int32),
            pltpu.VMEM((K,), jnp.int32),
            pltpu.VMEM((K + 16,), jnp.float32),
            pltpu.VMEM((K + 16,), jnp.int32),
            pltpu.VMEM((K,), jnp.int32),
            pltpu.VMEM((K + 16,), jnp.float32),
            pltpu.VMEM((K, INPUT_DIM), jnp.float32),
            pltpu.VMEM((K, INPUT_DIM), jnp.float32),
            pltpu.VMEM((ROWS_PER_W, INPUT_DIM), jnp.float32),
            pltpu.SemaphoreType.DMA,
            pltpu.SemaphoreType.DMA,
            pltpu.SemaphoreType.DMA,
            pltpu.SemaphoreType.DMA,
            pltpu.SemaphoreType.DMA,
        ],
    )(old_p, inputs_p, if_p, subr, subc, subv, supr, supc, supv, bounds)


def kernel(inputs, old_activation, input_fields, sub_rows, sub_cols, sub_vals,
           sup_rows, sup_cols, sup_vals):
    i32 = jnp.int32
    input_fields = input_fields.astype(i32)
    sub_rows = sub_rows.astype(i32)
    sub_cols = sub_cols.astype(i32)
    sup_rows = sup_rows.astype(i32)
    sup_cols = sup_cols.astype(i32)

    inputs_p = jnp.pad(inputs, ((0, B_PAD - B), (0, 0)))
    if_p = jnp.pad(input_fields, (0, B_PAD - B))
    subr_p = jnp.pad(sub_rows, (0, EDGE_PAD))
    subc_p = jnp.pad(sub_cols, (0, EDGE_PAD))
    subv_p = jnp.pad(sub_vals, (0, EDGE_PAD))
    supr_p = jnp.pad(sup_rows, (0, EDGE_PAD))
    supc_p = jnp.pad(sup_cols, (0, EDGE_PAD))
    supv_p = jnp.pad(sup_vals, (0, EDGE_PAD))

    row_starts = jnp.arange(NW + 1, dtype=i32) * ROWS_PER_W
    bs = jnp.searchsorted(sub_rows, row_starts).astype(i32)
    bp = jnp.searchsorted(sup_rows, row_starts).astype(i32)
    bounds = jnp.concatenate(
        [bs[:NW].reshape(2, 16), bs[1:NW + 1].reshape(2, 16),
         bp[:NW].reshape(2, 16), bp[1:NW + 1].reshape(2, 16)], axis=0)

    out_p, _ = _fused_kernel(old_activation, inputs_p, if_p, subr_p, subc_p, subv_p,
                             supr_p, supc_p, supv_p, bounds)
    return out_p[:NUM_DATA]
